# baseline (speedup 1.0000x reference)
import functools

import jax
import jax.numpy as jnp
from jax import lax
from jax.experimental import pallas as pl
from jax.experimental.pallas import tpu as pltpu
from jax.experimental.pallas import tpu_sc as plsc

N = 10000
E = 320000
NUM_RELS = 90
SUBMAT = 32

NC = 2
NS = 16
NW = NC * NS

NPT = 313
NLAST = N - (NW - 1) * NPT

CS = 128
PCH = 20
NPH = E // (PCH * CS)

CAPK = 1792
GC = 128

TN = 2000
NT = N // TN


def _y_body(w_ref, x_ref, y_ref):
    w = w_ref[...]
    wcat = jnp.concatenate([w, w, w, w], axis=1)
    ri = lax.broadcasted_iota(jnp.int32, (128, 128), 0)
    ci = lax.broadcasted_iota(jnp.int32, (128, 128), 1)
    wbd = jnp.where((ri // SUBMAT) == (ci // SUBMAT), wcat, 0.0)
    y_ref[...] = jnp.dot(x_ref[...], wbd, preferred_element_type=jnp.float32)


def _make_y(x, wr):
    return pl.pallas_call(
        _y_body,
        grid=(NT, NUM_RELS),
        in_specs=[
            pl.BlockSpec((128, 32), lambda nt, r: (r, 0)),
            pl.BlockSpec((TN, 128), lambda nt, r: (nt, 0)),
        ],
        out_specs=pl.BlockSpec((TN, 128), lambda nt, r: (r * NT + nt, 0)),
        out_shape=jax.ShapeDtypeStruct((NUM_RELS * N, 128), jnp.float32),
    )(wr, x)


def _sc_body(y_hbm, e3_hbm, out_hbm,
             st_a, st_b, idx_c, dloc_c, h2d,
             rows_a, rows_b, sem_sa, sem_sb, sem_a, sem_b):
    cid = lax.axis_index("c")
    sid = lax.axis_index("s")
    wid = cid * NS + sid
    lo = wid * NPT

    zf = jnp.zeros((16,), jnp.float32)
    def _z(r, c):
        for k in range(8):
            h2d.at[r][pl.ds(k * 16, 16)] = zf
        return c
    lax.fori_loop(0, NPT + 1, _z, 0)

    def _stage(p, st, sem):
        pltpu.async_copy(e3_hbm.at[p], st, sem)

    def _stage_wait(st, sem):
        pltpu.make_async_copy(e3_hbm.at[0], st, sem).wait()

    def _scan(st, ptrs):
        d = st.at[0]
        s = st.at[1]
        t = st.at[2]
        def _row(j, ptrs):
            out = []
            for k in range(8):
                ptr = ptrs[k]
                sl = pl.ds(k * 16, 16)
                dv = d.at[j][sl]
                m = (dv >= lo) & (dv < lo + NPT)
                idx = t.at[j][sl] * N + s.at[j][sl]
                pos = (jnp.broadcast_to(k * CAPK + ptr - 1, (16,))
                       + plsc.cumsum(m.astype(jnp.int32)))
                plsc.store_scatter(idx_c, [pos], idx, mask=m)
                plsc.store_scatter(dloc_c, [pos], dv - lo, mask=m)
                out.append(ptr + plsc.all_reduce_population_count(m)[0])
            return tuple(out)
        return lax.fori_loop(0, PCH, _row, ptrs)

    _stage(0, st_a, sem_sa)
    _stage(1, st_b, sem_sb)
    def _pair(i, ptrs):
        _stage_wait(st_a, sem_sa)
        ptrs = _scan(st_a, ptrs)
        @pl.when(2 * i + 2 < NPH)
        def _():
            _stage(2 * i + 2, st_a, sem_sa)
        _stage_wait(st_b, sem_sb)
        ptrs = _scan(st_b, ptrs)
        @pl.when(2 * i + 3 < NPH)
        def _():
            _stage(2 * i + 3, st_b, sem_sb)
        return ptrs
    zero = jnp.int32(0)
    cnts = lax.fori_loop(0, NPH // 2, _pair, (zero,) * 8)
    _stage_wait(st_a, sem_sa)
    cnts = _scan(st_a, cnts)

    zi = jnp.zeros((16,), jnp.int32)
    di = jnp.full((16,), NPT, jnp.int32)
    lane = lax.broadcasted_iota(jnp.int32, (16,), 0)
    for k in range(8):
        def _p(i, c, k=k):
            pos = jnp.broadcast_to(k * CAPK + cnts[k] + i * 16, (16,)) + lane
            plsc.store_scatter(idx_c, [pos], zi)
            plsc.store_scatter(dloc_c, [pos], di)
            return c
        lax.fori_loop(0, 8, _p, 0)

    def _g(k, c, rows, sem):
        pltpu.async_copy(
            y_hbm.at[idx_c.at[pl.ds(k * CAPK + c * GC, GC)]], rows, sem)

    def _gw(rows, sem):
        pltpu.make_async_copy(y_hbm.at[idx_c.at[pl.ds(0, GC)]], rows, sem).wait()

    def _acc(k, c, rows):
        cb = k * CAPK + c * GC
        def _grp(g, cc):
            dv = dloc_c[pl.ds(cb + g * 16, 16)]
            for l in range(16):
                dloc = dv[l]
                j = g * 16 + l
                for kk in range(8):
                    sl = pl.ds(kk * 16, 16)
                    plsc.addupdate(h2d.at[dloc, sl], rows.at[j][sl])
            return cc
        lax.fori_loop(0, GC // 16, _grp, 0)

    for k in range(8):
        nch = (cnts[k] + 127) // 128
        @pl.when(nch > 0)
        def _(k=k):
            _g(k, 0, rows_a, sem_a)
        @pl.when(nch > 1)
        def _(k=k):
            _g(k, 1, rows_b, sem_b)
        def _gpair(i, c, k=k, nch=nch):
            c0 = 2 * i
            _gw(rows_a, sem_a)
            _acc(k, c0, rows_a)
            @pl.when(c0 + 2 < nch)
            def _():
                _g(k, c0 + 2, rows_a, sem_a)
            _gw(rows_b, sem_b)
            _acc(k, c0 + 1, rows_b)
            @pl.when(c0 + 3 < nch)
            def _():
                _g(k, c0 + 3, rows_b, sem_b)
            return c
        lax.fori_loop(0, nch // 2, _gpair, 0)
        @pl.when(nch % 2 == 1)
        def _(k=k, nch=nch):
            _gw(rows_a, sem_a)
            _acc(k, nch - 1, rows_a)

    @pl.when(wid < NW - 1)
    def _():
        pltpu.sync_copy(h2d.at[pl.ds(0, NPT)], out_hbm.at[pl.ds(lo, NPT)])
    @pl.when(wid == NW - 1)
    def _():
        pltpu.sync_copy(h2d.at[pl.ds(0, NLAST)], out_hbm.at[pl.ds(lo, NLAST)])


def _make_sc(y, e3):
    mesh = plsc.VectorSubcoreMesh(core_axis_name="c", subcore_axis_name="s")
    f = pl.kernel(
        _sc_body,
        out_type=jax.ShapeDtypeStruct((N, 128), jnp.float32),
        mesh=mesh,
        compiler_params=pltpu.CompilerParams(
            use_tc_tiling_on_sc=False, needs_layout_passes=False),
        scratch_types=[
            pltpu.VMEM((3, PCH, CS), jnp.int32),
            pltpu.VMEM((3, PCH, CS), jnp.int32),
            pltpu.VMEM((8 * CAPK,), jnp.int32),
            pltpu.VMEM((8 * CAPK,), jnp.int32),
            pltpu.VMEM((NPT + 1, 128), jnp.float32),
            pltpu.VMEM((GC, 128), jnp.float32),
            pltpu.VMEM((GC, 128), jnp.float32),
            pltpu.SemaphoreType.DMA,
            pltpu.SemaphoreType.DMA,
            pltpu.SemaphoreType.DMA,
            pltpu.SemaphoreType.DMA,
        ],
    )
    return f(y, e3)


def _fin_body(h_ref, x_ref, norm_ref, lw_ref, b_ref, o_ref):
    lm = jnp.dot(x_ref[...], lw_ref[...], preferred_element_type=jnp.float32)
    o_ref[...] = h_ref[...] * norm_ref[...] + b_ref[...] + lm


def _make_fin(h, x, norm, loop_weight, bias2):
    return pl.pallas_call(
        _fin_body,
        grid=(NT,),
        in_specs=[
            pl.BlockSpec((TN, 128), lambda i: (i, 0)),
            pl.BlockSpec((TN, 128), lambda i: (i, 0)),
            pl.BlockSpec((TN, 1), lambda i: (i, 0)),
            pl.BlockSpec((128, 128), lambda i: (0, 0)),
            pl.BlockSpec((1, 128), lambda i: (0, 0)),
        ],
        out_specs=pl.BlockSpec((TN, 128), lambda i: (i, 0)),
        out_shape=jax.ShapeDtypeStruct((N, 128), jnp.float32),
    )(h, x, norm, loop_weight, bias2)


def kernel(x, edge_index, edge_type, norm, weight, loop_weight, bias_parm):
    wr = weight.reshape(NUM_RELS * 128, 32)
    src_r = edge_index[0].reshape(NPH, PCH, CS)
    dst_r = edge_index[1].reshape(NPH, PCH, CS)
    typ_r = edge_type.reshape(NPH, PCH, CS)
    e3 = jnp.stack([dst_r, src_r, typ_r], axis=1)

    y = _make_y(x, wr)
    h = _make_sc(y, e3)
    return _make_fin(h, x, norm, loop_weight, bias_parm.reshape(1, 128))


# speedup vs baseline: 1.6691x; 1.6691x over previous
import functools

import jax
import jax.numpy as jnp
from jax import lax
from jax.experimental import pallas as pl
from jax.experimental.pallas import tpu as pltpu
from jax.experimental.pallas import tpu_sc as plsc

N = 10000
E = 320000
NUM_RELS = 90
SUBMAT = 32

NC = 2
NS = 16
NW = NC * NS

NPT = 313
NLAST = N - (NW - 1) * NPT
DIVM = 13401
DIVS = 22
PAD_DST = 10016

CS = 128
APCH = 20
APH = 4
EPW = APH * APCH * CS
E_PAD = NW * EPW

NOWN = 33
BCAP = 448
NCTR = 48

CAP = 12288
GC = 128

TN = 2000
NT = N // TN

_SC_PARAMS = pltpu.CompilerParams(
    use_tc_tiling_on_sc=False, needs_layout_passes=False)


def _y_body(w_ref, x_ref, y_ref):
    w = w_ref[...]
    wcat = jnp.concatenate([w, w, w, w], axis=1)
    ri = lax.broadcasted_iota(jnp.int32, (128, 128), 0)
    ci = lax.broadcasted_iota(jnp.int32, (128, 128), 1)
    wbd = jnp.where((ri // SUBMAT) == (ci // SUBMAT), wcat, 0.0)
    y_ref[...] = jnp.dot(x_ref[...], wbd, preferred_element_type=jnp.float32)


def _make_y(x, wr):
    return pl.pallas_call(
        _y_body,
        grid=(NT, NUM_RELS),
        in_specs=[
            pl.BlockSpec((128, 32), lambda nt, r: (r, 0)),
            pl.BlockSpec((TN, 128), lambda nt, r: (nt, 0)),
        ],
        out_specs=pl.BlockSpec((TN, 128), lambda nt, r: (r * NT + nt, 0)),
        out_shape=jax.ShapeDtypeStruct((NUM_RELS * N, 128), jnp.float32),
    )(wr, x)


def _sca_body(e3_hbm, ridx_hbm, rdlo_hbm, cnts_hbm,
              st_a, st_b, rb_idx, rb_dlo, ctr, sem_sa, sem_sb):
    cid = lax.axis_index("c")
    sid = lax.axis_index("s")
    wid = cid * NS + sid
    lane = lax.broadcasted_iota(jnp.int32, (16,), 0)
    zi = jnp.zeros((16,), jnp.int32)

    for g in range(NCTR // 16):
        plsc.store_scatter(ctr, [lane + g * 16], zi)

    def _scan(st):
        d = st.at[0]
        s = st.at[1]
        t = st.at[2]
        def _row(j, c):
            for k in range(8):
                sl = pl.ds(k * 16, 16)
                dv = d.at[j][sl]
                idx = t.at[j][sl] * N + s.at[j][sl]
                o = lax.shift_right_logical(dv * DIVM, DIVS)
                dloc = dv - o * NPT
                cntv, lastm = plsc.scan_count(o)
                base = plsc.load_gather(ctr, [o])
                col = base + cntv - 1
                plsc.store_scatter(rb_idx, [o, col], idx)
                plsc.store_scatter(rb_dlo, [o, col], dloc)
                plsc.store_scatter(ctr, [o], base + cntv, mask=lastm)
            return c
        lax.fori_loop(0, APCH, _row, 0)

    my = e3_hbm.at[wid]
    pltpu.async_copy(my.at[0], st_a, sem_sa)
    pltpu.async_copy(my.at[1], st_b, sem_sb)
    pltpu.make_async_copy(my.at[0], st_a, sem_sa).wait()
    _scan(st_a)
    pltpu.async_copy(my.at[2], st_a, sem_sa)
    pltpu.make_async_copy(my.at[0], st_b, sem_sb).wait()
    _scan(st_b)
    pltpu.async_copy(my.at[3], st_b, sem_sb)
    pltpu.make_async_copy(my.at[0], st_a, sem_sa).wait()
    _scan(st_a)
    pltpu.make_async_copy(my.at[0], st_b, sem_sb).wait()
    _scan(st_b)

    pltpu.sync_copy(rb_idx, ridx_hbm.at[wid])
    pltpu.sync_copy(rb_dlo, rdlo_hbm.at[wid])
    pltpu.sync_copy(ctr, cnts_hbm.at[wid])


def _make_sca(e3p):
    mesh = plsc.VectorSubcoreMesh(core_axis_name="c", subcore_axis_name="s")
    f = pl.kernel(
        _sca_body,
        out_type=(
            jax.ShapeDtypeStruct((NW, NOWN, BCAP), jnp.int32),
            jax.ShapeDtypeStruct((NW, NOWN, BCAP), jnp.int32),
            jax.ShapeDtypeStruct((NW, NCTR), jnp.int32),
        ),
        mesh=mesh,
        compiler_params=_SC_PARAMS,
        scratch_types=[
            pltpu.VMEM((3, APCH, CS), jnp.int32),
            pltpu.VMEM((3, APCH, CS), jnp.int32),
            pltpu.VMEM((NOWN, BCAP), jnp.int32),
            pltpu.VMEM((NOWN, BCAP), jnp.int32),
            pltpu.VMEM((NCTR,), jnp.int32),
            pltpu.SemaphoreType.DMA,
            pltpu.SemaphoreType.DMA,
        ],
    )
    return f(e3p)


def _scb_body(y_hbm, ridx_hbm, rdlo_hbm, cnts_hbm, out_hbm,
              sidx, sdlo, cbuf, idx_c, dloc_c, h2d, rows_a, rows_b,
              sem_s, sem_a, sem_b):
    cid = lax.axis_index("c")
    sid = lax.axis_index("s")
    wid = cid * NS + sid
    lo = wid * NPT
    lane = lax.broadcasted_iota(jnp.int32, (16,), 0)

    zf = jnp.zeros((16,), jnp.float32)
    def _z(r, c):
        for k in range(8):
            h2d.at[r][pl.ds(k * 16, 16)] = zf
        return c
    lax.fori_loop(0, NPT + 1, _z, 0)

    pltpu.async_copy(cnts_hbm, cbuf, sem_s)
    for s in range(NW):
        pltpu.async_copy(ridx_hbm.at[s, wid], sidx.at[s], sem_s)
        pltpu.async_copy(rdlo_hbm.at[s, wid], sdlo.at[s], sem_s)
    pltpu.make_async_copy(cnts_hbm, cbuf, sem_s).wait()
    for s in range(NW):
        pltpu.make_async_copy(ridx_hbm.at[0, 0], sidx.at[s], sem_s).wait()
        pltpu.make_async_copy(ridx_hbm.at[0, 0], sdlo.at[s], sem_s).wait()

    wvec = jnp.broadcast_to(wid, (16,))
    clo = plsc.load_gather(cbuf, [lane, wvec])
    chi = plsc.load_gather(cbuf, [lane + 16, wvec])

    P = jnp.int32(0)
    for s in range(NW):
        def _cp(g, c, s=s, P=P):
            pos = jnp.broadcast_to(P + g * 16, (16,)) + lane
            plsc.store_scatter(idx_c, [pos], sidx.at[s][pl.ds(g * 16, 16)])
            plsc.store_scatter(dloc_c, [pos], sdlo.at[s][pl.ds(g * 16, 16)])
            return c
        lax.fori_loop(0, BCAP // 16, _cp, 0)
        P = P + (clo[s] if s < 16 else chi[s - 16])
    cnt = P

    zi = jnp.zeros((16,), jnp.int32)
    di = jnp.full((16,), NPT, jnp.int32)
    def _pd(i, c):
        pos = jnp.broadcast_to(cnt + i * 16, (16,)) + lane
        plsc.store_scatter(idx_c, [pos], zi)
        plsc.store_scatter(dloc_c, [pos], di)
        return c
    lax.fori_loop(0, 8, _pd, 0)
    nch = (cnt + 127) // 128

    def _g(c, rows, sem):
        pltpu.async_copy(y_hbm.at[idx_c.at[pl.ds(c * GC, GC)]], rows, sem)

    def _gw(rows, sem):
        pltpu.make_async_copy(y_hbm.at[idx_c.at[pl.ds(0, GC)]], rows, sem).wait()

    def _acc(c, rows):
        cb = c * GC
        def _grp(g, cc):
            dv = dloc_c[pl.ds(cb + g * 16, 16)]
            for l in range(16):
                dloc = dv[l]
                j = g * 16 + l
                for kk in range(8):
                    sl = pl.ds(kk * 16, 16)
                    plsc.addupdate(h2d.at[dloc, sl], rows.at[j][sl])
            return cc
        lax.fori_loop(0, GC // 16, _grp, 0)

    @pl.when(nch > 0)
    def _():
        _g(0, rows_a, sem_a)
    @pl.when(nch > 1)
    def _():
        _g(1, rows_b, sem_b)
    def _gpair(i, c):
        c0 = 2 * i
        _gw(rows_a, sem_a)
        _acc(c0, rows_a)
        @pl.when(c0 + 2 < nch)
        def _():
            _g(c0 + 2, rows_a, sem_a)
        _gw(rows_b, sem_b)
        _acc(c0 + 1, rows_b)
        @pl.when(c0 + 3 < nch)
        def _():
            _g(c0 + 3, rows_b, sem_b)
        return c
    lax.fori_loop(0, nch // 2, _gpair, 0)
    @pl.when(nch % 2 == 1)
    def _():
        _gw(rows_a, sem_a)
        _acc(nch - 1, rows_a)

    @pl.when(wid < NW - 1)
    def _():
        pltpu.sync_copy(h2d.at[pl.ds(0, NPT)], out_hbm.at[pl.ds(lo, NPT)])
    @pl.when(wid == NW - 1)
    def _():
        pltpu.sync_copy(h2d.at[pl.ds(0, NLAST)], out_hbm.at[pl.ds(lo, NLAST)])


def _make_scb(y, ridx, rdlo, cnts):
    mesh = plsc.VectorSubcoreMesh(core_axis_name="c", subcore_axis_name="s")
    f = pl.kernel(
        _scb_body,
        out_type=jax.ShapeDtypeStruct((N, 128), jnp.float32),
        mesh=mesh,
        compiler_params=_SC_PARAMS,
        scratch_types=[
            pltpu.VMEM((NW, BCAP), jnp.int32),
            pltpu.VMEM((NW, BCAP), jnp.int32),
            pltpu.VMEM((NW, NCTR), jnp.int32),
            pltpu.VMEM((CAP,), jnp.int32),
            pltpu.VMEM((CAP,), jnp.int32),
            pltpu.VMEM((NPT + 1, 128), jnp.float32),
            pltpu.VMEM((GC, 128), jnp.float32),
            pltpu.VMEM((GC, 128), jnp.float32),
            pltpu.SemaphoreType.DMA,
            pltpu.SemaphoreType.DMA,
            pltpu.SemaphoreType.DMA,
        ],
    )
    return f(y, ridx, rdlo, cnts)


def _fin_body(h_ref, x_ref, norm_ref, lw_ref, b_ref, o_ref):
    lm = jnp.dot(x_ref[...], lw_ref[...], preferred_element_type=jnp.float32)
    o_ref[...] = h_ref[...] * norm_ref[...] + b_ref[...] + lm


def _make_fin(h, x, norm, loop_weight, bias2):
    return pl.pallas_call(
        _fin_body,
        grid=(NT,),
        in_specs=[
            pl.BlockSpec((TN, 128), lambda i: (i, 0)),
            pl.BlockSpec((TN, 128), lambda i: (i, 0)),
            pl.BlockSpec((TN, 1), lambda i: (i, 0)),
            pl.BlockSpec((128, 128), lambda i: (0, 0)),
            pl.BlockSpec((1, 128), lambda i: (0, 0)),
        ],
        out_specs=pl.BlockSpec((TN, 128), lambda i: (i, 0)),
        out_shape=jax.ShapeDtypeStruct((N, 128), jnp.float32),
    )(h, x, norm, loop_weight, bias2)


def kernel(x, edge_index, edge_type, norm, weight, loop_weight, bias_parm):
    wr = weight.reshape(NUM_RELS * 128, 32)
    epr = E // NW
    padw = EPW - epr
    zpad = jnp.zeros((NW, padw), jnp.int32)
    src_p = jnp.concatenate([edge_index[0].reshape(NW, epr), zpad], axis=1)
    dst_p = jnp.concatenate(
        [edge_index[1].reshape(NW, epr),
         jnp.full((NW, padw), PAD_DST, jnp.int32)], axis=1)
    typ_p = jnp.concatenate([edge_type.reshape(NW, epr), zpad], axis=1)
    e3p = jnp.stack(
        [dst_p.reshape(NW, APH, APCH, CS),
         src_p.reshape(NW, APH, APCH, CS),
         typ_p.reshape(NW, APH, APCH, CS)], axis=2)

    y = _make_y(x, wr)
    ridx, rdlo, cnts = _make_sca(e3p)
    h = _make_scb(y, ridx, rdlo, cnts)
    return _make_fin(h, x, norm, loop_weight, bias_parm.reshape(1, 128))


# speedup vs baseline: 2.2047x; 1.3209x over previous
import functools

import jax
import jax.numpy as jnp
from jax import lax
from jax.experimental import pallas as pl
from jax.experimental.pallas import tpu as pltpu
from jax.experimental.pallas import tpu_sc as plsc

N = 10000
E = 320000
NUM_RELS = 90
SUBMAT = 32

NC = 2
NS = 16
NW = NC * NS

NPT = 313
NLAST = N - (NW - 1) * NPT
DIVM = 13401
DIVS = 22
PAD_DST = 10016

CS = 128
APCH = 20
APH = 4
EPW = APH * APCH * CS
E_PAD = NW * EPW

NOWN = 33
BCAP = 448
NCTR = 48

CAP = 12288
GC = 128

TN = 2000
NT = N // TN

_SC_PARAMS = pltpu.CompilerParams(
    use_tc_tiling_on_sc=False, needs_layout_passes=False)


def _y_body(w_ref, x_ref, y_ref):
    w = w_ref[...]
    wcat = jnp.concatenate([w, w, w, w], axis=1)
    ri = lax.broadcasted_iota(jnp.int32, (128, 128), 0)
    ci = lax.broadcasted_iota(jnp.int32, (128, 128), 1)
    wbd = jnp.where((ri // SUBMAT) == (ci // SUBMAT), wcat, 0.0)
    y_ref[...] = jnp.dot(x_ref[...], wbd, preferred_element_type=jnp.float32)


def _make_y(x, wr):
    return pl.pallas_call(
        _y_body,
        grid=(NT, NUM_RELS),
        in_specs=[
            pl.BlockSpec((128, 32), lambda nt, r: (r, 0)),
            pl.BlockSpec((TN, 128), lambda nt, r: (nt, 0)),
        ],
        out_specs=pl.BlockSpec((TN, 128), lambda nt, r: (r * NT + nt, 0)),
        out_shape=jax.ShapeDtypeStruct((NUM_RELS * N, 128), jnp.float32),
    )(wr, x)


def _sca_body(e3_hbm, ridx_hbm, rdlo_hbm, cnts_hbm,
              st_a, st_b, rb_idx, rb_dlo, ctr, sem_sa, sem_sb):
    cid = lax.axis_index("c")
    sid = lax.axis_index("s")
    wid = cid * NS + sid
    lane = lax.broadcasted_iota(jnp.int32, (16,), 0)
    zi = jnp.zeros((16,), jnp.int32)

    for g in range(NCTR // 16):
        plsc.store_scatter(ctr, [lane + g * 16], zi)

    def _scan(st):
        d = st.at[0]
        s = st.at[1]
        t = st.at[2]
        def _row(j, c):
            for k in range(8):
                sl = pl.ds(k * 16, 16)
                dv = d.at[j][sl]
                idx = t.at[j][sl] * N + s.at[j][sl]
                o = lax.shift_right_logical(dv * DIVM, DIVS)
                dloc = dv - o * NPT
                cntv, lastm = plsc.scan_count(o)
                base = plsc.load_gather(ctr, [o])
                col = base + cntv - 1
                plsc.store_scatter(rb_idx, [o, col], idx)
                plsc.store_scatter(rb_dlo, [o, col], dloc)
                plsc.store_scatter(ctr, [o], base + cntv, mask=lastm)
            return c
        lax.fori_loop(0, APCH, _row, 0)

    my = e3_hbm.at[wid]
    pltpu.async_copy(my.at[0], st_a, sem_sa)
    pltpu.async_copy(my.at[1], st_b, sem_sb)
    pltpu.make_async_copy(my.at[0], st_a, sem_sa).wait()
    _scan(st_a)
    pltpu.async_copy(my.at[2], st_a, sem_sa)
    pltpu.make_async_copy(my.at[0], st_b, sem_sb).wait()
    _scan(st_b)
    pltpu.async_copy(my.at[3], st_b, sem_sb)
    pltpu.make_async_copy(my.at[0], st_a, sem_sa).wait()
    _scan(st_a)
    pltpu.make_async_copy(my.at[0], st_b, sem_sb).wait()
    _scan(st_b)

    pltpu.sync_copy(rb_idx, ridx_hbm.at[wid])
    pltpu.sync_copy(rb_dlo, rdlo_hbm.at[wid])
    pltpu.sync_copy(ctr, cnts_hbm.at[wid])


def _make_sca(e3p):
    mesh = plsc.VectorSubcoreMesh(core_axis_name="c", subcore_axis_name="s")
    f = pl.kernel(
        _sca_body,
        out_type=(
            jax.ShapeDtypeStruct((NW, NOWN, BCAP), jnp.int32),
            jax.ShapeDtypeStruct((NW, NOWN, BCAP), jnp.int32),
            jax.ShapeDtypeStruct((NW, NCTR), jnp.int32),
        ),
        mesh=mesh,
        compiler_params=_SC_PARAMS,
        scratch_types=[
            pltpu.VMEM((3, APCH, CS), jnp.int32),
            pltpu.VMEM((3, APCH, CS), jnp.int32),
            pltpu.VMEM((NOWN, BCAP), jnp.int32),
            pltpu.VMEM((NOWN, BCAP), jnp.int32),
            pltpu.VMEM((NCTR,), jnp.int32),
            pltpu.SemaphoreType.DMA,
            pltpu.SemaphoreType.DMA,
        ],
    )
    return f(e3p)


def _scb_body(y_hbm, ridx_hbm, rdlo_hbm, cnts_hbm, z_hbm, out_hbm,
              sidx, sdlo, cbuf, idx_c, dloc2, h_sh, rows_a, rows_b,
              sem_s, sem_a, sem_b):
    cid = lax.axis_index("c")
    sid = lax.axis_index("s")
    wid = cid * NS + sid
    lo = wid * NPT
    srow = sid * (NPT + 1)
    lane = lax.broadcasted_iota(jnp.int32, (16,), 0)

    pltpu.sync_copy(z_hbm, h_sh.at[pl.ds(srow, NPT + 1)])

    pltpu.async_copy(cnts_hbm, cbuf, sem_s)
    for s in range(NW):
        pltpu.async_copy(ridx_hbm.at[s, wid], sidx.at[s], sem_s)
        pltpu.async_copy(rdlo_hbm.at[s, wid], sdlo.at[s], sem_s)
    pltpu.make_async_copy(cnts_hbm, cbuf, sem_s).wait()
    for s in range(NW):
        pltpu.make_async_copy(ridx_hbm.at[0, 0], sidx.at[s], sem_s).wait()
        pltpu.make_async_copy(ridx_hbm.at[0, 0], sdlo.at[s], sem_s).wait()

    wvec = jnp.broadcast_to(wid, (16,))
    clo = plsc.load_gather(cbuf, [lane, wvec])
    chi = plsc.load_gather(cbuf, [lane + 16, wvec])

    P = jnp.int32(0)
    for s in range(NW):
        def _cp(g, c, s=s, P=P):
            pos = jnp.broadcast_to(P + g * 16, (16,)) + lane
            plsc.store_scatter(idx_c, [pos], sidx.at[s][pl.ds(g * 16, 16)])
            plsc.store_scatter(
                dloc2, [pos >> 7, pos & 127],
                sdlo.at[s][pl.ds(g * 16, 16)] + jnp.broadcast_to(srow, (16,)))
            return c
        lax.fori_loop(0, BCAP // 16, _cp, 0)
        P = P + (clo[s] if s < 16 else chi[s - 16])
    cnt = P

    zi = jnp.zeros((16,), jnp.int32)
    def _pd(i, c):
        pos = jnp.broadcast_to(cnt + i * 16, (16,)) + lane
        plsc.store_scatter(idx_c, [pos], zi)
        plsc.store_scatter(dloc2, [pos >> 7, pos & 127],
                          jnp.broadcast_to(srow + NPT, (16,)))
        return c
    lax.fori_loop(0, 8, _pd, 0)
    nch = (cnt + 127) // 128

    def _g(c, rows, sem):
        pltpu.async_copy(y_hbm.at[idx_c.at[pl.ds(c * GC, GC)]], rows, sem)

    def _gw(rows, sem):
        pltpu.make_async_copy(y_hbm.at[idx_c.at[pl.ds(0, GC)]], rows, sem).wait()

    def _acc(c, rows):
        pltpu.sync_copy(rows, h_sh.at[dloc2.at[c]], add=True)

    @pl.when(nch > 0)
    def _():
        _g(0, rows_a, sem_a)
    @pl.when(nch > 1)
    def _():
        _g(1, rows_b, sem_b)
    def _gpair(i, c):
        c0 = 2 * i
        _gw(rows_a, sem_a)
        _acc(c0, rows_a)
        @pl.when(c0 + 2 < nch)
        def _():
            _g(c0 + 2, rows_a, sem_a)
        _gw(rows_b, sem_b)
        _acc(c0 + 1, rows_b)
        @pl.when(c0 + 3 < nch)
        def _():
            _g(c0 + 3, rows_b, sem_b)
        return c
    lax.fori_loop(0, nch // 2, _gpair, 0)
    @pl.when(nch % 2 == 1)
    def _():
        _gw(rows_a, sem_a)
        _acc(nch - 1, rows_a)

    @pl.when(wid < NW - 1)
    def _():
        pltpu.sync_copy(h_sh.at[pl.ds(srow, NPT)], out_hbm.at[pl.ds(lo, NPT)])
    @pl.when(wid == NW - 1)
    def _():
        pltpu.sync_copy(h_sh.at[pl.ds(srow, NLAST)],
                        out_hbm.at[pl.ds(lo, NLAST)])


def _make_scb(y, ridx, rdlo, cnts, zblk):
    mesh = plsc.VectorSubcoreMesh(core_axis_name="c", subcore_axis_name="s")
    f = pl.kernel(
        _scb_body,
        out_type=jax.ShapeDtypeStruct((N, 128), jnp.float32),
        mesh=mesh,
        compiler_params=_SC_PARAMS,
        scratch_types=[
            pltpu.VMEM((NW, BCAP), jnp.int32),
            pltpu.VMEM((NW, BCAP), jnp.int32),
            pltpu.VMEM((NW, NCTR), jnp.int32),
            pltpu.VMEM((CAP,), jnp.int32),
            pltpu.VMEM((CAP // GC, GC), jnp.int32),
            pltpu.VMEM_SHARED((NS * (NPT + 1), 128), jnp.float32),
            pltpu.VMEM((GC, 128), jnp.float32),
            pltpu.VMEM((GC, 128), jnp.float32),
            pltpu.SemaphoreType.DMA,
            pltpu.SemaphoreType.DMA,
            pltpu.SemaphoreType.DMA,
        ],
    )
    return f(y, ridx, rdlo, cnts, zblk)


def _fin_body(h_ref, x_ref, norm_ref, lw_ref, b_ref, o_ref):
    lm = jnp.dot(x_ref[...], lw_ref[...], preferred_element_type=jnp.float32)
    o_ref[...] = h_ref[...] * norm_ref[...] + b_ref[...] + lm


def _make_fin(h, x, norm, loop_weight, bias2):
    return pl.pallas_call(
        _fin_body,
        grid=(NT,),
        in_specs=[
            pl.BlockSpec((TN, 128), lambda i: (i, 0)),
            pl.BlockSpec((TN, 128), lambda i: (i, 0)),
            pl.BlockSpec((TN, 1), lambda i: (i, 0)),
            pl.BlockSpec((128, 128), lambda i: (0, 0)),
            pl.BlockSpec((1, 128), lambda i: (0, 0)),
        ],
        out_specs=pl.BlockSpec((TN, 128), lambda i: (i, 0)),
        out_shape=jax.ShapeDtypeStruct((N, 128), jnp.float32),
    )(h, x, norm, loop_weight, bias2)


def kernel(x, edge_index, edge_type, norm, weight, loop_weight, bias_parm):
    wr = weight.reshape(NUM_RELS * 128, 32)
    epr = E // NW
    padw = EPW - epr
    zpad = jnp.zeros((NW, padw), jnp.int32)
    src_p = jnp.concatenate([edge_index[0].reshape(NW, epr), zpad], axis=1)
    dst_p = jnp.concatenate(
        [edge_index[1].reshape(NW, epr),
         jnp.full((NW, padw), PAD_DST, jnp.int32)], axis=1)
    typ_p = jnp.concatenate([edge_type.reshape(NW, epr), zpad], axis=1)
    e3p = jnp.stack(
        [dst_p.reshape(NW, APH, APCH, CS),
         src_p.reshape(NW, APH, APCH, CS),
         typ_p.reshape(NW, APH, APCH, CS)], axis=2)

    y = _make_y(x, wr)
    ridx, rdlo, cnts = _make_sca(e3p)
    zblk = jnp.zeros((NPT + 1, 128), jnp.float32)
    h = _make_scb(y, ridx, rdlo, cnts, zblk)
    return _make_fin(h, x, norm, loop_weight, bias_parm.reshape(1, 128))
